# baseline (speedup 1.0000x reference)
import functools

import jax
import jax.numpy as jnp
from jax.experimental import pallas as pl
from jax.experimental.pallas import tpu as pltpu

_C = 80
_K = 300
_T = 256
_HM = 32


def _resize_mat():
    return jax.image.resize(jnp.eye(_HM, dtype=jnp.float32), (_T, _HM),
                            method="bilinear")


_QB = 8


def _split2(x):
    hi = x.astype(jnp.bfloat16)
    lo = (x - hi.astype(jnp.float32)).astype(jnp.bfloat16)
    return hi, lo


def _mask_body(qidx_ref, *refs):
    mask_refs = refs[:_QB]
    abd_ref, at_ref, out_ref = refs[_QB], refs[_QB + 1], refs[_QB + 2]
    mstack = jnp.concatenate([r[0, 0] for r in mask_refs], axis=0)
    mhi, mlo = _split2(mstack)
    v = (jax.lax.dot(abd_ref[...], mhi, preferred_element_type=jnp.float32) +
         jax.lax.dot(abd_ref[...], mlo, preferred_element_type=jnp.float32))
    vhi, vlo = _split2(v)
    r = (jax.lax.dot(vhi, at_ref[...], preferred_element_type=jnp.float32) +
         jax.lax.dot(vlo, at_ref[...], preferred_element_type=jnp.float32))
    out_ref[0] = (r > 0.0).astype(jnp.float32).reshape(_QB, _T, _T)


def _box_body(qidx3_ref, box_ref, tmat_ref, out_ref):
    qvec = qidx3_ref[0]
    iot = jax.lax.broadcasted_iota(jnp.int32, (1000, _K), 0)
    onehot = (iot == qvec).astype(jnp.float32)
    gathered = jax.lax.dot_general(
        onehot, box_ref[0], (((0,), (0,)), ((), ())),
        precision=jax.lax.Precision.HIGHEST,
        preferred_element_type=jnp.float32)
    out_ref[0] = jax.lax.dot(gathered, tmat_ref[0],
                             precision=jax.lax.Precision.HIGHEST,
                             preferred_element_type=jnp.float32)


def kernel(pred_logits, pred_boxes, pred_masks, orig_target_sizes):
    b_dim, q_dim = pred_logits.shape[0], pred_logits.shape[1]

    scores_all = jax.nn.sigmoid(pred_logits).reshape(b_dim, q_dim * _C)
    scores, index_flat = jax.lax.top_k(scores_all, _K)
    labels = index_flat % _C
    qidx = (index_flat // _C).astype(jnp.int32)

    a_mat = _resize_mat()
    at_mat = a_mat.T

    n_steps = (_K + _QB - 1) // _QB
    qidx_pad = jnp.pad(qidx, ((0, 0), (0, n_steps * _QB - _K)))

    def _gather_spec(g):
        return pl.BlockSpec(
            (1, 1, _HM, _HM),
            lambda b, j, qref, g=g: (b, qref[b, j * _QB + g], 0, 0))

    masks_out = pl.pallas_call(
        _mask_body,
        grid_spec=pltpu.PrefetchScalarGridSpec(
            num_scalar_prefetch=1,
            grid=(b_dim, n_steps),
            in_specs=(
                [_gather_spec(g) for g in range(_QB)] + [
                    pl.BlockSpec((_QB * _T, _QB * _HM),
                                 lambda b, j, qref: (0, 0)),
                    pl.BlockSpec((_HM, _T), lambda b, j, qref: (0, 0)),
                ]),
            out_specs=pl.BlockSpec((1, _QB, _T, _T),
                                   lambda b, j, qref: (b, j, 0, 0)),
        ),
        out_shape=jax.ShapeDtypeStruct((b_dim, _K, _T, _T), jnp.float32),
    )(qidx_pad, *([pred_masks] * _QB),
      jnp.kron(jnp.eye(_QB, dtype=jnp.float32), a_mat).astype(jnp.bfloat16),
      at_mat.astype(jnp.bfloat16))

    wh = orig_target_sizes.astype(jnp.float32)
    s = jnp.concatenate([wh, wh], axis=1)
    base = jnp.array([[1.0, 0.0, 1.0, 0.0],
                      [0.0, 1.0, 0.0, 1.0],
                      [-0.5, 0.0, 0.5, 0.0],
                      [0.0, -0.5, 0.0, 0.5]], jnp.float32)
    tmat = base[None, :, :] * s[:, None, :]

    boxes_out = pl.pallas_call(
        _box_body,
        grid=(b_dim,),
        in_specs=[
            pl.BlockSpec((1, 1, _K), lambda b: (b, 0, 0)),
            pl.BlockSpec((1, q_dim, 4), lambda b: (b, 0, 0)),
            pl.BlockSpec((1, 4, 4), lambda b: (b, 0, 0)),
        ],
        out_specs=pl.BlockSpec((1, _K, 4), lambda b: (b, 0, 0)),
        out_shape=jax.ShapeDtypeStruct((b_dim, _K, 4), jnp.float32),
    )(qidx.reshape(b_dim, 1, _K), pred_boxes, tmat)

    return scores, labels, boxes_out, masks_out


# speedup vs baseline: 1.1800x; 1.1800x over previous
import functools

import jax
import jax.numpy as jnp
from jax.experimental import pallas as pl
from jax.experimental.pallas import tpu as pltpu
from jax.experimental.pallas import tpu_sc as plsc

_C = 80
_K = 300
_T = 256
_HM = 32


def _resize_mat():
    return jax.image.resize(jnp.eye(_HM, dtype=jnp.float32), (_T, _HM),
                            method="bilinear")


_QB = 8


def _split2(x):
    hi = x.astype(jnp.bfloat16)
    lo = (x - hi.astype(jnp.float32)).astype(jnp.bfloat16)
    return hi, lo


def _sc_gather_masks(masks_flat, gidx_flat):
    info = plsc.get_sparse_core_info()
    n_cores = info.num_cores
    nw = n_cores * info.num_subcores
    n = gidx_flat.shape[0]
    b_per_w = n // nw
    mesh = plsc.VectorSubcoreMesh(core_axis_name="c", subcore_axis_name="s")

    @functools.partial(
        pl.kernel, mesh=mesh,
        out_type=jax.ShapeDtypeStruct((n, _HM * _HM), jnp.float32),
        scratch_types=[
            pltpu.VMEM((b_per_w,), jnp.int32),
            pltpu.VMEM((b_per_w, _HM * _HM), jnp.float32),
            pltpu.SemaphoreType.DMA,
        ],
    )
    def gather_kernel(table_hbm, idx_hbm, out_hbm, idx_v, rows_v, sem):
        wid = jax.lax.axis_index("s") * n_cores + jax.lax.axis_index("c")
        base = wid * b_per_w
        pltpu.sync_copy(idx_hbm.at[pl.ds(base, b_per_w)], idx_v)
        pltpu.async_copy(table_hbm.at[idx_v], rows_v, sem).wait()
        pltpu.sync_copy(rows_v, out_hbm.at[pl.ds(base, b_per_w)])

    return gather_kernel(masks_flat, gidx_flat)


def _mask_body(mask_ref, a_ref, at_ref, out_ref):
    mstack = jnp.concatenate([mask_ref[0, g] for g in range(_QB)], axis=1)
    mhi, mlo = _split2(mstack)
    v = (jax.lax.dot(a_ref[...], mhi, preferred_element_type=jnp.float32) +
         jax.lax.dot(a_ref[...], mlo, preferred_element_type=jnp.float32))
    for g in range(_QB):
        vhi, vlo = _split2(v[:, g * _HM:(g + 1) * _HM])
        r = (jax.lax.dot(vhi, at_ref[...], preferred_element_type=jnp.float32) +
             jax.lax.dot(vlo, at_ref[...], preferred_element_type=jnp.float32))
        out_ref[0, g] = (r > 0.0).astype(jnp.float32)


def _box_body(qidx3_ref, box_ref, tmat_ref, out_ref):
    qvec = qidx3_ref[0]
    iot = jax.lax.broadcasted_iota(jnp.int32, (1000, _K), 0)
    onehot = (iot == qvec).astype(jnp.float32)
    gathered = jax.lax.dot_general(
        onehot, box_ref[0], (((0,), (0,)), ((), ())),
        precision=jax.lax.Precision.HIGHEST,
        preferred_element_type=jnp.float32)
    out_ref[0] = jax.lax.dot(gathered, tmat_ref[0],
                             precision=jax.lax.Precision.HIGHEST,
                             preferred_element_type=jnp.float32)


def kernel(pred_logits, pred_boxes, pred_masks, orig_target_sizes):
    b_dim, q_dim = pred_logits.shape[0], pred_logits.shape[1]

    scores_all = jax.nn.sigmoid(pred_logits).reshape(b_dim, q_dim * _C)
    scores, index_flat = jax.lax.top_k(scores_all, _K)
    labels = index_flat % _C
    qidx = (index_flat // _C).astype(jnp.int32)

    a_mat = _resize_mat()
    at_mat = a_mat.T

    n_steps = (_K + _QB - 1) // _QB
    kq = n_steps * _QB

    gidx = qidx + (jnp.arange(b_dim, dtype=jnp.int32) * q_dim)[:, None]
    gidx_b = jnp.pad(gidx, ((0, 0), (0, kq - _K)))
    n_total = b_dim * kq
    info = plsc.get_sparse_core_info()
    align = 8 * info.num_cores * info.num_subcores
    n_pad = ((n_total + align - 1) // align) * align
    gidx_flat = jnp.pad(gidx_b.reshape(-1), (0, n_pad - n_total))
    compact = _sc_gather_masks(
        pred_masks.reshape(b_dim * q_dim, _HM * _HM), gidx_flat)
    compact = compact[:n_total].reshape(b_dim, kq, _HM, _HM)

    masks_out = pl.pallas_call(
        _mask_body,
        grid=(b_dim, n_steps),
        in_specs=[
            pl.BlockSpec((1, _QB, _HM, _HM), lambda b, j: (b, j, 0, 0)),
            pl.BlockSpec((_T, _HM), lambda b, j: (0, 0)),
            pl.BlockSpec((_HM, _T), lambda b, j: (0, 0)),
        ],
        out_specs=pl.BlockSpec((1, _QB, _T, _T), lambda b, j: (b, j, 0, 0)),
        out_shape=jax.ShapeDtypeStruct((b_dim, _K, _T, _T), jnp.float32),
    )(compact, a_mat.astype(jnp.bfloat16), at_mat.astype(jnp.bfloat16))

    wh = orig_target_sizes.astype(jnp.float32)
    s = jnp.concatenate([wh, wh], axis=1)
    base = jnp.array([[1.0, 0.0, 1.0, 0.0],
                      [0.0, 1.0, 0.0, 1.0],
                      [-0.5, 0.0, 0.5, 0.0],
                      [0.0, -0.5, 0.0, 0.5]], jnp.float32)
    tmat = base[None, :, :] * s[:, None, :]

    boxes_out = pl.pallas_call(
        _box_body,
        grid=(b_dim,),
        in_specs=[
            pl.BlockSpec((1, 1, _K), lambda b: (b, 0, 0)),
            pl.BlockSpec((1, q_dim, 4), lambda b: (b, 0, 0)),
            pl.BlockSpec((1, 4, 4), lambda b: (b, 0, 0)),
        ],
        out_specs=pl.BlockSpec((1, _K, 4), lambda b: (b, 0, 0)),
        out_shape=jax.ShapeDtypeStruct((b_dim, _K, 4), jnp.float32),
    )(qidx.reshape(b_dim, 1, _K), pred_boxes, tmat)

    return scores, labels, boxes_out, masks_out


# speedup vs baseline: 1.2677x; 1.0743x over previous
import functools

import jax
import jax.numpy as jnp
from jax.experimental import pallas as pl
from jax.experimental.pallas import tpu as pltpu
from jax.experimental.pallas import tpu_sc as plsc

_C = 80
_K = 300
_T = 256
_HM = 32


def _resize_mat():
    return jax.image.resize(jnp.eye(_HM, dtype=jnp.float32), (_T, _HM),
                            method="bilinear")


_QB = 16


def _split2(x):
    hi = x.astype(jnp.bfloat16)
    lo = (x - hi.astype(jnp.float32)).astype(jnp.bfloat16)
    return hi, lo


def _sc_gather_masks(masks_flat, gidx_flat):
    info = plsc.get_sparse_core_info()
    n_cores = info.num_cores
    nw = n_cores * info.num_subcores
    n = gidx_flat.shape[0]
    b_per_w = n // nw
    mesh = plsc.VectorSubcoreMesh(core_axis_name="c", subcore_axis_name="s")

    @functools.partial(
        pl.kernel, mesh=mesh,
        out_type=jax.ShapeDtypeStruct((n, _HM * _HM), jnp.float32),
        scratch_types=[
            pltpu.VMEM((b_per_w,), jnp.int32),
            pltpu.VMEM((b_per_w, _HM * _HM), jnp.float32),
            pltpu.SemaphoreType.DMA,
        ],
    )
    def gather_kernel(table_hbm, idx_hbm, out_hbm, idx_v, rows_v, sem):
        wid = jax.lax.axis_index("s") * n_cores + jax.lax.axis_index("c")
        base = wid * b_per_w
        pltpu.sync_copy(idx_hbm.at[pl.ds(base, b_per_w)], idx_v)
        pltpu.async_copy(table_hbm.at[idx_v], rows_v, sem).wait()
        pltpu.sync_copy(rows_v, out_hbm.at[pl.ds(base, b_per_w)])

    return gather_kernel(masks_flat, gidx_flat)


def _mask_body(mask_ref, a_ref, at_ref, out_ref):
    mstack = jnp.concatenate([mask_ref[0, g] for g in range(_QB)], axis=1)
    mhi, mlo = _split2(mstack)
    v = (jax.lax.dot(a_ref[...], mhi, preferred_element_type=jnp.float32) +
         jax.lax.dot(a_ref[...], mlo, preferred_element_type=jnp.float32))
    for g in range(_QB):
        vhi, vlo = _split2(v[:, g * _HM:(g + 1) * _HM])
        r = (jax.lax.dot(vhi, at_ref[...], preferred_element_type=jnp.float32) +
             jax.lax.dot(vlo, at_ref[...], preferred_element_type=jnp.float32))
        out_ref[0, g] = (r > 0.0).astype(jnp.float32)


def _box_body(qidx3_ref, box_ref, tmat_ref, out_ref):
    qvec = qidx3_ref[0]
    iot = jax.lax.broadcasted_iota(jnp.int32, (1000, _K), 0)
    onehot = (iot == qvec).astype(jnp.float32)
    gathered = jax.lax.dot_general(
        onehot, box_ref[0], (((0,), (0,)), ((), ())),
        precision=jax.lax.Precision.HIGHEST,
        preferred_element_type=jnp.float32)
    out_ref[0] = jax.lax.dot(gathered, tmat_ref[0],
                             precision=jax.lax.Precision.HIGHEST,
                             preferred_element_type=jnp.float32)


def kernel(pred_logits, pred_boxes, pred_masks, orig_target_sizes):
    b_dim, q_dim = pred_logits.shape[0], pred_logits.shape[1]

    scores_all = jax.nn.sigmoid(pred_logits).reshape(b_dim, q_dim * _C)
    scores, index_flat = jax.lax.top_k(scores_all, _K)
    labels = index_flat % _C
    qidx = (index_flat // _C).astype(jnp.int32)

    a_mat = _resize_mat()
    at_mat = a_mat.T

    n_steps = (_K + _QB - 1) // _QB
    kq = n_steps * _QB

    gidx = qidx + (jnp.arange(b_dim, dtype=jnp.int32) * q_dim)[:, None]
    gidx_b = jnp.pad(gidx, ((0, 0), (0, kq - _K)))
    n_total = b_dim * kq
    info = plsc.get_sparse_core_info()
    align = 8 * info.num_cores * info.num_subcores
    n_pad = ((n_total + align - 1) // align) * align
    gidx_flat = jnp.pad(gidx_b.reshape(-1), (0, n_pad - n_total))
    compact = _sc_gather_masks(
        pred_masks.reshape(b_dim * q_dim, _HM * _HM), gidx_flat)
    compact = compact[:n_total].reshape(b_dim, kq, _HM, _HM)

    masks_out = pl.pallas_call(
        _mask_body,
        grid=(b_dim, n_steps),
        in_specs=[
            pl.BlockSpec((1, _QB, _HM, _HM), lambda b, j: (b, j, 0, 0)),
            pl.BlockSpec((_T, _HM), lambda b, j: (0, 0)),
            pl.BlockSpec((_HM, _T), lambda b, j: (0, 0)),
        ],
        out_specs=pl.BlockSpec((1, _QB, _T, _T), lambda b, j: (b, j, 0, 0)),
        out_shape=jax.ShapeDtypeStruct((b_dim, _K, _T, _T), jnp.float32),
    )(compact, a_mat.astype(jnp.bfloat16), at_mat.astype(jnp.bfloat16))

    wh = orig_target_sizes.astype(jnp.float32)
    s = jnp.concatenate([wh, wh], axis=1)
    base = jnp.array([[1.0, 0.0, 1.0, 0.0],
                      [0.0, 1.0, 0.0, 1.0],
                      [-0.5, 0.0, 0.5, 0.0],
                      [0.0, -0.5, 0.0, 0.5]], jnp.float32)
    tmat = base[None, :, :] * s[:, None, :]

    boxes_out = pl.pallas_call(
        _box_body,
        grid=(b_dim,),
        in_specs=[
            pl.BlockSpec((1, 1, _K), lambda b: (b, 0, 0)),
            pl.BlockSpec((1, q_dim, 4), lambda b: (b, 0, 0)),
            pl.BlockSpec((1, 4, 4), lambda b: (b, 0, 0)),
        ],
        out_specs=pl.BlockSpec((1, _K, 4), lambda b: (b, 0, 0)),
        out_shape=jax.ShapeDtypeStruct((b_dim, _K, 4), jnp.float32),
    )(qidx.reshape(b_dim, 1, _K), pred_boxes, tmat)

    return scores, labels, boxes_out, masks_out


# speedup vs baseline: 1.2948x; 1.0214x over previous
import functools

import jax
import jax.numpy as jnp
from jax.experimental import pallas as pl
from jax.experimental.pallas import tpu as pltpu
from jax.experimental.pallas import tpu_sc as plsc

_C = 80
_K = 300
_T = 256
_HM = 32


def _resize_mat():
    return jax.image.resize(jnp.eye(_HM, dtype=jnp.float32), (_T, _HM),
                            method="bilinear")


_QB = 32


def _split2(x):
    hi = x.astype(jnp.bfloat16)
    lo = (x - hi.astype(jnp.float32)).astype(jnp.bfloat16)
    return hi, lo


def _sc_gather_masks(masks_flat, gidx_flat):
    info = plsc.get_sparse_core_info()
    n_cores = info.num_cores
    nw = n_cores * info.num_subcores
    n = gidx_flat.shape[0]
    b_per_w = n // nw
    mesh = plsc.VectorSubcoreMesh(core_axis_name="c", subcore_axis_name="s")

    @functools.partial(
        pl.kernel, mesh=mesh,
        out_type=jax.ShapeDtypeStruct((n, _HM * _HM), jnp.float32),
        scratch_types=[
            pltpu.VMEM((b_per_w,), jnp.int32),
            pltpu.VMEM((b_per_w, _HM * _HM), jnp.float32),
            pltpu.SemaphoreType.DMA,
        ],
    )
    def gather_kernel(table_hbm, idx_hbm, out_hbm, idx_v, rows_v, sem):
        wid = jax.lax.axis_index("s") * n_cores + jax.lax.axis_index("c")
        base = wid * b_per_w
        pltpu.sync_copy(idx_hbm.at[pl.ds(base, b_per_w)], idx_v)
        pltpu.async_copy(table_hbm.at[idx_v], rows_v, sem).wait()
        pltpu.sync_copy(rows_v, out_hbm.at[pl.ds(base, b_per_w)])

    return gather_kernel(masks_flat, gidx_flat)


def _mask_body(mask_ref, a_ref, at_ref, out_ref):
    mstack = jnp.concatenate([mask_ref[0, g] for g in range(_QB)], axis=1)
    mhi, mlo = _split2(mstack)
    v = (jax.lax.dot(a_ref[...], mhi, preferred_element_type=jnp.float32) +
         jax.lax.dot(a_ref[...], mlo, preferred_element_type=jnp.float32))
    for g in range(_QB):
        vhi, vlo = _split2(v[:, g * _HM:(g + 1) * _HM])
        r = (jax.lax.dot(vhi, at_ref[...], preferred_element_type=jnp.float32) +
             jax.lax.dot(vlo, at_ref[...], preferred_element_type=jnp.float32))
        out_ref[0, g] = (r > 0.0).astype(jnp.float32)


def _box_body(qidx3_ref, box_ref, tmat_ref, out_ref):
    qvec = qidx3_ref[0]
    iot = jax.lax.broadcasted_iota(jnp.int32, (1000, _K), 0)
    onehot = (iot == qvec).astype(jnp.float32)
    gathered = jax.lax.dot_general(
        onehot, box_ref[0], (((0,), (0,)), ((), ())),
        precision=jax.lax.Precision.HIGHEST,
        preferred_element_type=jnp.float32)
    out_ref[0] = jax.lax.dot(gathered, tmat_ref[0],
                             precision=jax.lax.Precision.HIGHEST,
                             preferred_element_type=jnp.float32)


def kernel(pred_logits, pred_boxes, pred_masks, orig_target_sizes):
    b_dim, q_dim = pred_logits.shape[0], pred_logits.shape[1]

    scores_all = jax.nn.sigmoid(pred_logits).reshape(b_dim, q_dim * _C)
    scores, index_flat = jax.lax.top_k(scores_all, _K)
    labels = index_flat % _C
    qidx = (index_flat // _C).astype(jnp.int32)

    a_mat = _resize_mat()
    at_mat = a_mat.T

    n_steps = (_K + _QB - 1) // _QB
    kq = n_steps * _QB

    gidx = qidx + (jnp.arange(b_dim, dtype=jnp.int32) * q_dim)[:, None]
    gidx_b = jnp.pad(gidx, ((0, 0), (0, kq - _K)))
    n_total = b_dim * kq
    info = plsc.get_sparse_core_info()
    align = 8 * info.num_cores * info.num_subcores
    n_pad = ((n_total + align - 1) // align) * align
    gidx_flat = jnp.pad(gidx_b.reshape(-1), (0, n_pad - n_total))
    compact = _sc_gather_masks(
        pred_masks.reshape(b_dim * q_dim, _HM * _HM), gidx_flat)
    compact = compact[:n_total].reshape(b_dim, kq, _HM, _HM)

    masks_out = pl.pallas_call(
        _mask_body,
        grid=(b_dim, n_steps),
        in_specs=[
            pl.BlockSpec((1, _QB, _HM, _HM), lambda b, j: (b, j, 0, 0)),
            pl.BlockSpec((_T, _HM), lambda b, j: (0, 0)),
            pl.BlockSpec((_HM, _T), lambda b, j: (0, 0)),
        ],
        out_specs=pl.BlockSpec((1, _QB, _T, _T), lambda b, j: (b, j, 0, 0)),
        out_shape=jax.ShapeDtypeStruct((b_dim, _K, _T, _T), jnp.float32),
    )(compact, a_mat.astype(jnp.bfloat16), at_mat.astype(jnp.bfloat16))

    wh = orig_target_sizes.astype(jnp.float32)
    s = jnp.concatenate([wh, wh], axis=1)
    base = jnp.array([[1.0, 0.0, 1.0, 0.0],
                      [0.0, 1.0, 0.0, 1.0],
                      [-0.5, 0.0, 0.5, 0.0],
                      [0.0, -0.5, 0.0, 0.5]], jnp.float32)
    tmat = base[None, :, :] * s[:, None, :]

    boxes_out = pl.pallas_call(
        _box_body,
        grid=(b_dim,),
        in_specs=[
            pl.BlockSpec((1, 1, _K), lambda b: (b, 0, 0)),
            pl.BlockSpec((1, q_dim, 4), lambda b: (b, 0, 0)),
            pl.BlockSpec((1, 4, 4), lambda b: (b, 0, 0)),
        ],
        out_specs=pl.BlockSpec((1, _K, 4), lambda b: (b, 0, 0)),
        out_shape=jax.ShapeDtypeStruct((b_dim, _K, 4), jnp.float32),
    )(qidx.reshape(b_dim, 1, _K), pred_boxes, tmat)

    return scores, labels, boxes_out, masks_out


# speedup vs baseline: 1.3088x; 1.0109x over previous
import functools

import jax
import jax.numpy as jnp
from jax.experimental import pallas as pl
from jax.experimental.pallas import tpu as pltpu
from jax.experimental.pallas import tpu_sc as plsc

_C = 80
_K = 300
_T = 256
_HM = 32


def _resize_mat():
    return jax.image.resize(jnp.eye(_HM, dtype=jnp.float32), (_T, _HM),
                            method="bilinear")


_QB = 64


def _split2(x):
    hi = x.astype(jnp.bfloat16)
    lo = (x - hi.astype(jnp.float32)).astype(jnp.bfloat16)
    return hi, lo


def _sc_gather_masks(masks_flat, gidx_flat):
    info = plsc.get_sparse_core_info()
    n_cores = info.num_cores
    nw = n_cores * info.num_subcores
    n = gidx_flat.shape[0]
    b_per_w = n // nw
    mesh = plsc.VectorSubcoreMesh(core_axis_name="c", subcore_axis_name="s")

    @functools.partial(
        pl.kernel, mesh=mesh,
        out_type=jax.ShapeDtypeStruct((n, _HM * _HM), jnp.float32),
        scratch_types=[
            pltpu.VMEM((b_per_w,), jnp.int32),
            pltpu.VMEM((b_per_w, _HM * _HM), jnp.float32),
            pltpu.SemaphoreType.DMA,
        ],
    )
    def gather_kernel(table_hbm, idx_hbm, out_hbm, idx_v, rows_v, sem):
        wid = jax.lax.axis_index("s") * n_cores + jax.lax.axis_index("c")
        base = wid * b_per_w
        pltpu.sync_copy(idx_hbm.at[pl.ds(base, b_per_w)], idx_v)
        pltpu.async_copy(table_hbm.at[idx_v], rows_v, sem).wait()
        pltpu.sync_copy(rows_v, out_hbm.at[pl.ds(base, b_per_w)])

    return gather_kernel(masks_flat, gidx_flat)


def _mask_body(mask_ref, a_ref, at_ref, out_ref):
    mstack = jnp.concatenate([mask_ref[0, g] for g in range(_QB)], axis=1)
    mhi, mlo = _split2(mstack)
    v = (jax.lax.dot(a_ref[...], mhi, preferred_element_type=jnp.float32) +
         jax.lax.dot(a_ref[...], mlo, preferred_element_type=jnp.float32))
    for g in range(_QB):
        vhi, vlo = _split2(v[:, g * _HM:(g + 1) * _HM])
        r = (jax.lax.dot(vhi, at_ref[...], preferred_element_type=jnp.float32) +
             jax.lax.dot(vlo, at_ref[...], preferred_element_type=jnp.float32))
        out_ref[0, g] = (r > 0.0).astype(jnp.float32)


def _box_body(qidx3_ref, box_ref, tmat_ref, out_ref):
    qvec = qidx3_ref[0]
    iot = jax.lax.broadcasted_iota(jnp.int32, (1000, _K), 0)
    onehot = (iot == qvec).astype(jnp.float32)
    gathered = jax.lax.dot_general(
        onehot, box_ref[0], (((0,), (0,)), ((), ())),
        precision=jax.lax.Precision.HIGHEST,
        preferred_element_type=jnp.float32)
    out_ref[0] = jax.lax.dot(gathered, tmat_ref[0],
                             precision=jax.lax.Precision.HIGHEST,
                             preferred_element_type=jnp.float32)


def kernel(pred_logits, pred_boxes, pred_masks, orig_target_sizes):
    b_dim, q_dim = pred_logits.shape[0], pred_logits.shape[1]

    scores_all = jax.nn.sigmoid(pred_logits).reshape(b_dim, q_dim * _C)
    scores, index_flat = jax.lax.top_k(scores_all, _K)
    labels = index_flat % _C
    qidx = (index_flat // _C).astype(jnp.int32)

    a_mat = _resize_mat()
    at_mat = a_mat.T

    n_steps = (_K + _QB - 1) // _QB
    kq = n_steps * _QB

    gidx = qidx + (jnp.arange(b_dim, dtype=jnp.int32) * q_dim)[:, None]
    gidx_b = jnp.pad(gidx, ((0, 0), (0, kq - _K)))
    n_total = b_dim * kq
    info = plsc.get_sparse_core_info()
    align = 8 * info.num_cores * info.num_subcores
    n_pad = ((n_total + align - 1) // align) * align
    gidx_flat = jnp.pad(gidx_b.reshape(-1), (0, n_pad - n_total))
    compact = _sc_gather_masks(
        pred_masks.reshape(b_dim * q_dim, _HM * _HM), gidx_flat)
    compact = compact[:n_total].reshape(b_dim, kq, _HM, _HM)

    masks_out = pl.pallas_call(
        _mask_body,
        grid=(b_dim, n_steps),
        in_specs=[
            pl.BlockSpec((1, _QB, _HM, _HM), lambda b, j: (b, j, 0, 0)),
            pl.BlockSpec((_T, _HM), lambda b, j: (0, 0)),
            pl.BlockSpec((_HM, _T), lambda b, j: (0, 0)),
        ],
        out_specs=pl.BlockSpec((1, _QB, _T, _T), lambda b, j: (b, j, 0, 0)),
        out_shape=jax.ShapeDtypeStruct((b_dim, _K, _T, _T), jnp.float32),
    )(compact, a_mat.astype(jnp.bfloat16), at_mat.astype(jnp.bfloat16))

    wh = orig_target_sizes.astype(jnp.float32)
    s = jnp.concatenate([wh, wh], axis=1)
    base = jnp.array([[1.0, 0.0, 1.0, 0.0],
                      [0.0, 1.0, 0.0, 1.0],
                      [-0.5, 0.0, 0.5, 0.0],
                      [0.0, -0.5, 0.0, 0.5]], jnp.float32)
    tmat = base[None, :, :] * s[:, None, :]

    boxes_out = pl.pallas_call(
        _box_body,
        grid=(b_dim,),
        in_specs=[
            pl.BlockSpec((1, 1, _K), lambda b: (b, 0, 0)),
            pl.BlockSpec((1, q_dim, 4), lambda b: (b, 0, 0)),
            pl.BlockSpec((1, 4, 4), lambda b: (b, 0, 0)),
        ],
        out_specs=pl.BlockSpec((1, _K, 4), lambda b: (b, 0, 0)),
        out_shape=jax.ShapeDtypeStruct((b_dim, _K, 4), jnp.float32),
    )(qidx.reshape(b_dim, 1, _K), pred_boxes, tmat)

    return scores, labels, boxes_out, masks_out
